# baseline (speedup 1.0000x reference)
import functools

import jax
import jax.numpy as jnp
from jax import lax
from jax.experimental import pallas as pl
from jax.experimental.pallas import tpu as pltpu
from jax.experimental.pallas import tpu_sc as plsc

_VOCAB = 100000
_EMB = 32
_WIN = 20
_BATCH = 1024

_NC = 2
_NS = 16
_NW = _NC * _NS
_BPW = _BATCH // _NW
_IPW = _BPW * _WIN
_CHUNK = 128
_NCHUNK = _IPW // _CHUNK


def _make_pooling_kernel():
    mesh = plsc.VectorSubcoreMesh(
        core_axis_name="c", subcore_axis_name="s",
        num_cores=_NC, num_subcores=_NS,
    )

    @functools.partial(
        pl.kernel,
        mesh=mesh,
        compiler_params=pltpu.CompilerParams(use_tc_tiling_on_sc=False),
        out_type=jax.ShapeDtypeStruct((_BATCH, _EMB), jnp.float32),
        scratch_types=[
            pltpu.VMEM((_NCHUNK, _CHUNK), jnp.int32),
            pltpu.VMEM((_IPW, _EMB), jnp.float32),
            pltpu.VMEM((_BPW, _EMB), jnp.float32),
            pltpu.SemaphoreType.DMA,
        ],
    )
    def pooling(ctx_hbm, table_hbm, pooled_hbm, idx_v, rows_v, pool_v, sem):
        wid = lax.axis_index("s") * _NC + lax.axis_index("c")
        pltpu.sync_copy(ctx_hbm.at[wid], idx_v)
        copies = [
            pltpu.async_copy(
                table_hbm.at[idx_v.at[c]],
                rows_v.at[pl.ds(c * _CHUNK, _CHUNK)],
                sem,
            )
            for c in range(_NCHUNK)
        ]
        for cp in copies:
            cp.wait()

        inv = jnp.float32(1.0 / _WIN)

        def pool_one(b, carry):
            p0 = b * _WIN
            a0 = rows_v[p0, 0:16]
            a1 = rows_v[p0, 16:32]
            for w in range(1, _WIN):
                a0 = a0 + rows_v[p0 + w, 0:16]
                a1 = a1 + rows_v[p0 + w, 16:32]
            pool_v[b, 0:16] = a0 * inv
            pool_v[b, 16:32] = a1 * inv
            return carry

        lax.fori_loop(0, _BPW, pool_one, 0)
        pltpu.sync_copy(pool_v, pooled_hbm.at[pl.ds(wid * _BPW, _BPW)])

    return pooling


@functools.lru_cache(maxsize=1)
def _get_pooling():
    return _make_pooling_kernel()

_TV = 2048


def _proj_body(pooled_ref, w_ref, b_ref, out_ref):
    acc = lax.dot_general(
        pooled_ref[...], w_ref[...],
        dimension_numbers=(((1,), (1,)), ((), ())),
        preferred_element_type=jnp.float32,
    )
    out_ref[...] = acc + b_ref[...]


def _project(pooled, W, b2d):
    return pl.pallas_call(
        _proj_body,
        grid=(pl.cdiv(_VOCAB, _TV),),
        in_specs=[
            pl.BlockSpec((_BATCH, _EMB), lambda j: (0, 0)),
            pl.BlockSpec((_TV, _EMB), lambda j: (j, 0)),
            pl.BlockSpec((1, _TV), lambda j: (0, j)),
        ],
        out_specs=pl.BlockSpec((_BATCH, _TV), lambda j: (0, j)),
        out_shape=jax.ShapeDtypeStruct((_BATCH, _VOCAB), jnp.float32),
    )(pooled, W, b2d)


def kernel(context, emb_table, W, b):
    ctx = context.astype(jnp.int32).reshape(_NW, _NCHUNK, _CHUNK)
    pooled = _get_pooling()(ctx, emb_table)
    return _project(pooled, W, b.reshape(1, _VOCAB))
